# baseline (speedup 1.0000x reference)
import functools

import jax
import jax.numpy as jnp
from jax import lax
from jax.experimental import pallas as pl
from jax.experimental.pallas import tpu as pltpu
from jax.experimental.pallas import tpu_sc as plsc

N = 10000
E = 320000
D = 128
G = 16
NPAD = 10240
ROWS = NPAD // 128
NC = 2
NS = 16
L = 16
NW = NC * NS
EPW = E // NW
EPT = E // NS
CHUNK = NPAD // NS
BATCH_PAD = 127
RSQRT_MAGIC = 0x5F3759DF


def _prep_tc_body(out_ref, z_ref, w_ref, batch_ref, h_ref, rm_ref, meta_ref):
  w_head = w_ref[0:D, :]
  w_last = w_ref[D, 0]
  h = jnp.dot(out_ref[...], w_head,
              preferred_element_type=jnp.float32)
  h = h.reshape(ROWS, 128)
  rm = (z_ref[...] != 100).astype(jnp.float32)
  h_ref[...] = h + rm * w_last
  rm_ref[...] = rm

  batch = batch_ref[...]
  bc = jnp.minimum(batch, G - 1)
  bmin = jnp.min(bc, axis=1, keepdims=True)
  bmax = jnp.max(bc, axis=1, keepdims=True)

  batch_f = batch.reshape(1, NPAD)
  gids = lax.broadcasted_iota(jnp.int32, (G, 1), 0)
  eq_f = jnp.where(batch_f == gids, 1.0, 0.0)
  counts_r = jnp.sum(eq_f, axis=1).reshape(1, G)
  li = lax.broadcasted_iota(jnp.int32, (G, G), 0)
  lj = lax.broadcasted_iota(jnp.int32, (G, G), 1)
  upper = jnp.where(li < lj, 1.0, 0.0)
  starts_r = jnp.dot(counts_r, upper,
                     preferred_element_type=jnp.float32)
  ends_r = starts_r + counts_r

  gids_r = lax.broadcasted_iota(jnp.int32, (1, G), 1)
  lo = jnp.sum(jnp.where(bmin == gids_r, starts_r, 0.0), axis=1,
               keepdims=True)
  end = jnp.sum(jnp.where(bmax == gids_r, ends_r, 0.0), axis=1,
                keepdims=True)
  meta_ref[:, 0:1] = jnp.floor(lo * (1.0 / 128.0)).astype(jnp.int32)
  meta_ref[:, 1:2] = jnp.floor((end + 127.0) * (1.0 / 128.0)).astype(jnp.int32)


def _prep_tc(out_p, z_p, w, batch_p):
  return pl.pallas_call(
      _prep_tc_body,
      out_shape=(
          jax.ShapeDtypeStruct((ROWS, 128), jnp.float32),
          jax.ShapeDtypeStruct((ROWS, 128), jnp.float32),
          jax.ShapeDtypeStruct((ROWS, 8), jnp.int32),
      ),
  )(out_p, z_p, w, batch_p)


def _mega_sc(row, col, h):
  @functools.partial(
      pl.kernel,
      out_type=(
          jax.ShapeDtypeStruct((NC, NPAD), jnp.float32),
          jax.ShapeDtypeStruct((NPAD,), jnp.float32),
      ),
      mesh=_sc_mesh(),
      compiler_params=pltpu.CompilerParams(needs_layout_passes=False),
      scratch_types=[
          pltpu.VMEM((EPT,), jnp.int32),
          pltpu.VMEM((EPW,), jnp.int32),
          pltpu.VMEM((EPW,), jnp.int32),
          pltpu.VMEM((NPAD,), jnp.float32),
          pltpu.VMEM((NS, CHUNK), jnp.float32),
          pltpu.VMEM((CHUNK,), jnp.float32),
          pltpu.VMEM((CHUNK,), jnp.float32),
          pltpu.VMEM((NPAD,), jnp.float32),
          pltpu.VMEM_SHARED((NS, NPAD), jnp.float32),
          pltpu.VMEM_SHARED((NPAD,), jnp.float32),
      ],
  )
  def body(row_hbm, col_hbm, h_hbm, agg_hbm, deg_hbm, cola_v, row_v, col_v,
           acc_v, red_v, h_v, uc_v, ufull_v, shared, shared_u):
    cid = lax.axis_index("c")
    sid = lax.axis_index("s")
    wid = cid * NS + sid
    off = sid * CHUNK
    zeros = jnp.zeros((L,), jnp.float32)
    ones = jnp.ones((L,), jnp.float32)

    pltpu.sync_copy(col_hbm.at[pl.ds(sid * EPT, EPT)], cola_v)

    def zbody(i, _):
      acc_v[pl.ds(i * L, L)] = zeros
      return 0
    lax.fori_loop(0, NPAD // L, zbody, 0)

    def dbody(i, _):
      idx = cola_v[pl.ds(i * L, L)]
      plsc.addupdate_scatter(acc_v, [idx], ones)
      return 0
    lax.fori_loop(0, EPT // L, dbody, 0)

    pltpu.sync_copy(acc_v, shared.at[sid])
    plsc.subcore_barrier()
    pltpu.sync_copy(shared.at[:, pl.ds(off, CHUNK)], red_v)

    def rbody(i, _):
      acc = red_v[0, pl.ds(i * L, L)]
      for j in range(1, NS):
        acc = acc + red_v[j, pl.ds(i * L, L)]
      red_v[0, pl.ds(i * L, L)] = acc
      return 0
    lax.fori_loop(0, CHUNK // L, rbody, 0)

    @pl.when(cid == 0)
    def _():
      pltpu.sync_copy(red_v.at[0], deg_hbm.at[pl.ds(off, CHUNK)])

    pltpu.sync_copy(h_hbm.at[pl.ds(off, CHUNK)], h_v)
    magic = jnp.full((L,), RSQRT_MAGIC, jnp.int32)

    def ubody(i, _):
      d = red_v[0, pl.ds(i * L, L)] + 1.0
      y = plsc.bitcast(magic - (plsc.bitcast(d, jnp.int32) >> 1), jnp.float32)
      for _ in range(4):
        y = y * (1.5 - 0.5 * d * y * y)
      uc_v[pl.ds(i * L, L)] = h_v[pl.ds(i * L, L)] * y
      return 0
    lax.fori_loop(0, CHUNK // L, ubody, 0)

    pltpu.sync_copy(uc_v, shared_u.at[pl.ds(off, CHUNK)])
    plsc.subcore_barrier()
    pltpu.sync_copy(shared_u, ufull_v)

    pltpu.sync_copy(row_hbm.at[pl.ds(wid * EPW, EPW)], row_v)
    pltpu.sync_copy(col_hbm.at[pl.ds(wid * EPW, EPW)], col_v)
    lax.fori_loop(0, NPAD // L, zbody, 0)

    def ebody(i, _):
      r = row_v[pl.ds(i * L, L)]
      c = col_v[pl.ds(i * L, L)]
      vals = plsc.load_gather(ufull_v, [r])
      plsc.addupdate_scatter(acc_v, [c], vals)
      return 0
    lax.fori_loop(0, EPW // L, ebody, 0)

    pltpu.sync_copy(acc_v, shared.at[sid])
    plsc.subcore_barrier()
    pltpu.sync_copy(shared.at[:, pl.ds(off, CHUNK)], red_v)
    lax.fori_loop(0, CHUNK // L, rbody, 0)
    pltpu.sync_copy(red_v.at[0], agg_hbm.at[cid, pl.ds(off, CHUNK)])

  return body(row, col, h)


def _sc_mesh():
  return plsc.VectorSubcoreMesh(core_axis_name="c", subcore_axis_name="s")


def _pool_tc_body(out_ref, rm_ref, batch_ref, h_ref, deg_ref, agg2_ref, b_ref,
                  meta_ref, o_ref, rank_ref, score_ref):
  dinv = lax.rsqrt(deg_ref[...] + 1.0)
  u = h_ref[...] * dinv
  agg = dinv * (agg2_ref[0] + agg2_ref[1] + u)
  score = jnp.tanh(agg + b_ref[0, 0])
  score_ref[...] = score

  batch = batch_ref[...]
  batch_f = batch.reshape(1, NPAD)
  gids = lax.broadcasted_iota(jnp.int32, (G, 1), 0)
  eq_f = jnp.where(batch_f == gids, 1.0, 0.0)
  counts = jnp.sum(eq_f, axis=1, keepdims=True)
  k = jnp.floor((counts + 1.0) * 0.5)
  k_node = jnp.sum(eq_f * k, axis=0).reshape(ROWS, 128)

  def rbody(rb, _):
    s_blk = score_ref[pl.ds(rb, 1), :].reshape(128, 1)
    b_blk = batch_ref[pl.ds(rb, 1), :].reshape(128, 1)
    i_blk = rb * 128 + lax.broadcasted_iota(jnp.int32, (128, 1), 0)
    lo = meta_ref[rb, 0]
    hi = meta_ref[rb, 1]

    def cbody(cb, acc):
      s_col = score_ref[pl.ds(cb, 1), :]
      b_col = batch_ref[pl.ds(cb, 1), :]
      i_col = cb * 128 + lax.broadcasted_iota(jnp.int32, (1, 128), 1)
      beats = jnp.where(
          (b_col == b_blk) & (
              (s_col > s_blk) | ((s_col == s_blk) & (i_col < i_blk))),
          1.0, 0.0)
      return acc + jnp.sum(beats, axis=1, keepdims=True)

    rank = lax.fori_loop(lo, hi, cbody, jnp.zeros((128, 1), jnp.float32))
    rank_ref[pl.ds(rb, 1), :] = rank.reshape(1, 128)
    return 0

  lax.fori_loop(0, ROWS, rbody, 0)
  sel = jnp.where(rank_ref[...] < k_node, 1.0, 0.0)
  sel_f = sel.reshape(1, NPAD)
  w_gate = (sel * score).reshape(1, NPAD)

  a = eq_f * w_gate
  sums = jnp.dot(a, out_ref[...], preferred_element_type=jnp.float32)
  rm_f = rm_ref[...].reshape(1, NPAD)
  rm_sum = jnp.sum(a * rm_f, axis=1, keepdims=True)
  nsel = jnp.sum(eq_f * sel_f, axis=1, keepdims=True)
  denom = jnp.maximum(nsel, 1.0)
  o_ref[:, 0:D] = sums / denom
  o_ref[:, D:D + 1] = rm_sum / denom


def _pool_tc(out_p, rm, batch_p, h, deg, agg2, b2, meta):
  return pl.pallas_call(
      _pool_tc_body,
      out_shape=jax.ShapeDtypeStruct((G, D + 1), jnp.float32),
      in_specs=[pl.BlockSpec(memory_space=pltpu.VMEM)] * 7
      + [pl.BlockSpec(memory_space=pltpu.SMEM)],
      scratch_shapes=[pltpu.VMEM((ROWS, 128), jnp.float32),
                      pltpu.VMEM((ROWS, 128), jnp.float32)],
  )(out_p, rm, batch_p, h, deg, agg2, b2, meta)


def kernel(out, z, edge_index, edge_attr, batch, W, b):
  del edge_attr
  row = edge_index[0]
  col = edge_index[1]

  out_p = jnp.pad(out, ((0, NPAD - N), (0, 0)))
  z_p = jnp.pad(z, (0, NPAD - N), constant_values=100).reshape(ROWS, 128)
  batch_p = jnp.pad(batch, (0, NPAD - N),
                    constant_values=BATCH_PAD).reshape(ROWS, 128)

  h, rm, meta = _prep_tc(out_p, z_p, W, batch_p)
  agg2, deg = _mega_sc(row, col, h.reshape(NPAD))
  return _pool_tc(out_p, rm, batch_p, h, deg.reshape(ROWS, 128),
                  agg2.reshape(2, ROWS, 128), b.reshape(1, 1), meta)


# speedup vs baseline: 1.1140x; 1.1140x over previous
import functools

import jax
import jax.numpy as jnp
from jax import lax
from jax.experimental import pallas as pl
from jax.experimental.pallas import tpu as pltpu
from jax.experimental.pallas import tpu_sc as plsc

N = 10000
E = 320000
D = 128
G = 16
NPAD = 10240
ROWS = NPAD // 128
NC = 2
NS = 16
L = 16
NW = NC * NS
EPW = E // NW
CHUNK = NPAD // NS
BATCH_PAD = 127


def _sc_mesh():
  return plsc.VectorSubcoreMesh(core_axis_name="c", subcore_axis_name="s")


def _deg_sc(edge_index):
  @functools.partial(
      pl.kernel,
      out_type=jax.ShapeDtypeStruct((NC, NPAD), jnp.float32),
      mesh=_sc_mesh(),
      compiler_params=pltpu.CompilerParams(needs_layout_passes=False),
      scratch_types=[
          pltpu.VMEM((EPW,), jnp.int32),
          pltpu.VMEM((NPAD,), jnp.float32),
          pltpu.VMEM((NS, CHUNK), jnp.float32),
          pltpu.VMEM_SHARED((NS, NPAD), jnp.float32),
      ],
  )
  def body(ei_hbm, out_hbm, col_v, acc_v, red_v, shared):
    cid = lax.axis_index("c")
    sid = lax.axis_index("s")
    wid = cid * NS + sid
    pltpu.sync_copy(ei_hbm.at[pl.ds(E + wid * EPW, EPW)], col_v)
    zeros = jnp.zeros((L,), jnp.float32)
    ones = jnp.ones((L,), jnp.float32)

    def zbody(i, _):
      acc_v[pl.ds(i * L, L)] = zeros
      return 0
    lax.fori_loop(0, NPAD // L, zbody, 0)

    def ebody(i, _):
      idx = col_v[pl.ds(i * L, L)]
      plsc.addupdate_scatter(acc_v, [idx], ones)
      return 0
    lax.fori_loop(0, EPW // L, ebody, 0)

    pltpu.sync_copy(acc_v, shared.at[sid])
    plsc.subcore_barrier()
    off = sid * CHUNK
    pltpu.sync_copy(shared.at[:, pl.ds(off, CHUNK)], red_v)

    def rbody(i, _):
      acc = red_v[0, pl.ds(i * L, L)]
      for j in range(1, NS):
        acc = acc + red_v[j, pl.ds(i * L, L)]
      red_v[0, pl.ds(i * L, L)] = acc
      return 0
    lax.fori_loop(0, CHUNK // L, rbody, 0)
    pltpu.sync_copy(red_v.at[0], out_hbm.at[cid, pl.ds(off, CHUNK)])

  return body(edge_index)


def _agg_sc(edge_index, u):
  @functools.partial(
      pl.kernel,
      out_type=jax.ShapeDtypeStruct((NC, NPAD), jnp.float32),
      mesh=_sc_mesh(),
      compiler_params=pltpu.CompilerParams(needs_layout_passes=False),
      scratch_types=[
          pltpu.VMEM((EPW,), jnp.int32),
          pltpu.VMEM((EPW,), jnp.int32),
          pltpu.VMEM((NPAD,), jnp.float32),
          pltpu.VMEM((NPAD,), jnp.float32),
          pltpu.VMEM((NS, CHUNK), jnp.float32),
          pltpu.VMEM_SHARED((NS, NPAD), jnp.float32),
      ],
  )
  def body(ei_hbm, u_hbm, out_hbm, row_v, col_v, u_v, acc_v, red_v,
           shared):
    cid = lax.axis_index("c")
    sid = lax.axis_index("s")
    wid = cid * NS + sid
    pltpu.sync_copy(ei_hbm.at[pl.ds(wid * EPW, EPW)], row_v)
    pltpu.sync_copy(ei_hbm.at[pl.ds(E + wid * EPW, EPW)], col_v)
    pltpu.sync_copy(u_hbm, u_v)
    zeros = jnp.zeros((L,), jnp.float32)

    def zbody(i, _):
      acc_v[pl.ds(i * L, L)] = zeros
      return 0
    lax.fori_loop(0, NPAD // L, zbody, 0)

    def ebody(i, _):
      r = row_v[pl.ds(i * L, L)]
      c = col_v[pl.ds(i * L, L)]
      vals = plsc.load_gather(u_v, [r])
      plsc.addupdate_scatter(acc_v, [c], vals)
      return 0
    lax.fori_loop(0, EPW // L, ebody, 0)

    pltpu.sync_copy(acc_v, shared.at[sid])
    plsc.subcore_barrier()
    off = sid * CHUNK
    pltpu.sync_copy(shared.at[:, pl.ds(off, CHUNK)], red_v)

    def rbody(i, _):
      acc = red_v[0, pl.ds(i * L, L)]
      for j in range(1, NS):
        acc = acc + red_v[j, pl.ds(i * L, L)]
      red_v[0, pl.ds(i * L, L)] = acc
      return 0
    lax.fori_loop(0, CHUNK // L, rbody, 0)
    pltpu.sync_copy(red_v.at[0], out_hbm.at[cid, pl.ds(off, CHUNK)])

  return body(edge_index, u)


def _prep_tc_body(out_ref, z_ref, w_ref, deg_ref, u_ref, dinv_ref, rm_ref):
  w_head = w_ref[0:D, :]
  w_last = w_ref[D, 0]
  h = jnp.dot(out_ref[...], w_head,
              preferred_element_type=jnp.float32)
  h = h.reshape(ROWS, 128)
  rm = (z_ref[...] != 100).astype(jnp.float32)
  h = h + rm * w_last
  deg = deg_ref[0] + deg_ref[1] + 1.0
  dinv = lax.rsqrt(deg)
  dinv_ref[...] = dinv
  u_ref[...] = h * dinv
  rm_ref[...] = rm


def _prep_tc(out_p, z_p, w, deg2):
  return pl.pallas_call(
      _prep_tc_body,
      out_shape=(
          jax.ShapeDtypeStruct((ROWS, 128), jnp.float32),
          jax.ShapeDtypeStruct((ROWS, 128), jnp.float32),
          jax.ShapeDtypeStruct((ROWS, 128), jnp.float32),
      ),
  )(out_p, z_p, w, deg2)


def _meta_tc_body(batch_ref, o_ref):
  batch = batch_ref[...]
  bc = jnp.minimum(batch, G - 1)
  bmin = jnp.min(bc, axis=1, keepdims=True)
  bmax = jnp.max(bc, axis=1, keepdims=True)

  batch_f = batch.reshape(1, NPAD)
  gids = lax.broadcasted_iota(jnp.int32, (G, 1), 0)
  eq_f = jnp.where(batch_f == gids, 1.0, 0.0)
  counts_r = jnp.sum(eq_f, axis=1).reshape(1, G)
  li = lax.broadcasted_iota(jnp.int32, (G, G), 0)
  lj = lax.broadcasted_iota(jnp.int32, (G, G), 1)
  upper = jnp.where(li < lj, 1.0, 0.0)
  starts_r = jnp.dot(counts_r, upper,
                     preferred_element_type=jnp.float32)
  ends_r = starts_r + counts_r

  gids_r = lax.broadcasted_iota(jnp.int32, (1, G), 1)
  lo = jnp.sum(jnp.where(bmin == gids_r, starts_r, 0.0), axis=1,
               keepdims=True)
  end = jnp.sum(jnp.where(bmax == gids_r, ends_r, 0.0), axis=1,
                keepdims=True)
  lo_cb = jnp.floor(lo * (1.0 / 128.0)).astype(jnp.int32)
  hi_cb = jnp.floor((end + 127.0) * (1.0 / 128.0)).astype(jnp.int32)
  o_ref[:, 0:1] = lo_cb
  o_ref[:, 1:2] = hi_cb


def _meta_tc(batch_p):
  return pl.pallas_call(
      _meta_tc_body,
      out_shape=jax.ShapeDtypeStruct((ROWS, 8), jnp.int32),
  )(batch_p)


def _pool_tc_body(out_ref, rm_ref, batch_ref, u_ref, dinv_ref, agg2_ref, b_ref,
                  meta_ref, o_ref, rank_ref, score_ref):
  u = u_ref[...]
  agg = dinv_ref[...] * (agg2_ref[0] + agg2_ref[1] + u)
  score = jnp.tanh(agg + b_ref[0, 0])
  score_ref[...] = score

  batch = batch_ref[...]
  batch_f = batch.reshape(1, NPAD)
  gids = lax.broadcasted_iota(jnp.int32, (G, 1), 0)
  eq_f = jnp.where(batch_f == gids, 1.0, 0.0)
  counts = jnp.sum(eq_f, axis=1, keepdims=True)
  k = jnp.floor((counts + 1.0) * 0.5)
  k_node = jnp.sum(eq_f * k, axis=0).reshape(ROWS, 128)

  def rbody(rb, _):
    s_blk = score_ref[pl.ds(rb, 1), :].reshape(128, 1)
    b_blk = batch_ref[pl.ds(rb, 1), :].reshape(128, 1)
    i_blk = rb * 128 + lax.broadcasted_iota(jnp.int32, (128, 1), 0)
    lo = meta_ref[rb, 0]
    hi = meta_ref[rb, 1]

    def cbody(cb, acc):
      s_col = score_ref[pl.ds(cb, 1), :]
      b_col = batch_ref[pl.ds(cb, 1), :]
      i_col = cb * 128 + lax.broadcasted_iota(jnp.int32, (1, 128), 1)
      beats = jnp.where(
          (b_col == b_blk) & (
              (s_col > s_blk) | ((s_col == s_blk) & (i_col < i_blk))),
          1.0, 0.0)
      return acc + jnp.sum(beats, axis=1, keepdims=True)

    rank = lax.fori_loop(lo, hi, cbody, jnp.zeros((128, 1), jnp.float32))
    rank_ref[pl.ds(rb, 1), :] = rank.reshape(1, 128)
    return 0

  lax.fori_loop(0, ROWS, rbody, 0)
  sel = jnp.where(rank_ref[...] < k_node, 1.0, 0.0)
  sel_f = sel.reshape(1, NPAD)
  w_gate = (sel * score).reshape(1, NPAD)

  a = eq_f * w_gate
  sums = jnp.dot(a, out_ref[...], preferred_element_type=jnp.float32)
  rm_f = rm_ref[...].reshape(1, NPAD)
  rm_sum = jnp.sum(a * rm_f, axis=1, keepdims=True)
  nsel = jnp.sum(eq_f * sel_f, axis=1, keepdims=True)
  denom = jnp.maximum(nsel, 1.0)
  o_ref[:, 0:D] = sums / denom
  o_ref[:, D:D + 1] = rm_sum / denom


def _pool_tc(out_p, rm, batch_p, u, dinv, agg2, b2, meta):
  return pl.pallas_call(
      _pool_tc_body,
      out_shape=jax.ShapeDtypeStruct((G, D + 1), jnp.float32),
      in_specs=[pl.BlockSpec(memory_space=pltpu.VMEM)] * 7
      + [pl.BlockSpec(memory_space=pltpu.SMEM)],
      scratch_shapes=[pltpu.VMEM((ROWS, 128), jnp.float32),
                      pltpu.VMEM((ROWS, 128), jnp.float32)],
  )(out_p, rm, batch_p, u, dinv, agg2, b2, meta)


def kernel(out, z, edge_index, edge_attr, batch, W, b):
  del edge_attr
  out_p = jnp.pad(out, ((0, NPAD - N), (0, 0)))
  z_p = jnp.pad(z, (0, NPAD - N), constant_values=100).reshape(ROWS, 128)
  batch_p = jnp.pad(batch, (0, NPAD - N),
                    constant_values=BATCH_PAD).reshape(ROWS, 128)

  meta = _meta_tc(batch_p)
  ei_flat = edge_index.reshape(2 * E)
  deg2 = _deg_sc(ei_flat)
  u, dinv, rm = _prep_tc(out_p, z_p, W,
                         deg2.reshape(2, ROWS, 128))
  agg2 = _agg_sc(ei_flat, u.reshape(NPAD))
  return _pool_tc(out_p, rm, batch_p, u, dinv,
                  agg2.reshape(2, ROWS, 128), b.reshape(1, 1), meta)
